# baseline (speedup 1.0000x reference)
import functools

import jax
import jax.numpy as jnp
from jax.experimental import pallas as pl

_F32 = jnp.float32
_BF16 = jnp.bfloat16
_Q = jnp.float8_e4m3fn
_QA = jnp.float4_e2m1fn

_BM = 400
_SAFETY = 1.1
_INV = 1.0 / _SAFETY


def _xq_kernel(x_ref, w_ref, b_ref, hq_ref, cs_ref, *, act):
    x = x_ref[...].astype(_F32)
    if act == "elu":
        x = jnp.where(x > 0, x, jnp.exp(x) - 1.0)
    y = jnp.dot(x.astype(_BF16), w_ref[...].astype(_BF16),
                preferred_element_type=_F32) + b_ref[...]
    cmax = jnp.maximum(jnp.max(jnp.abs(y), axis=0, keepdims=True), 1e-30)
    cs_ref[...] = cmax
    hq_ref[...] = (y * (1.0 / cmax)).astype(_Q)


def _xq(x, w, b, act):
    n, d_in = x.shape
    d_out = w.shape[1]
    return pl.pallas_call(
        functools.partial(_xq_kernel, act=act),
        out_shape=[
            jax.ShapeDtypeStruct((n, d_out), _Q),
            jax.ShapeDtypeStruct((1, d_out), _F32),
        ],
    )(x, w, b.reshape(1, d_out))


def _qprop_kernel(a_ref, hq_ref, aq_ref, rs_ref, hqn_ref):
    a = a_ref[...]
    rowmax = jnp.maximum(jnp.max(a, axis=1, keepdims=True), 1e-30)
    rs_ref[...] = rowmax
    aq = (a * (1.0 / rowmax)).astype(_QA)
    aq_ref[0] = aq
    acc = jax.lax.dot_general(aq, hq_ref[...], (((1,), (0,)), ((), ())),
                              preferred_element_type=_F32)
    hqn_ref[...] = (acc * (rowmax * _INV)).astype(_Q)


def _qprop(adj, hq):
    n = adj.shape[0]
    d = hq.shape[1]
    nb = n // _BM
    return pl.pallas_call(
        _qprop_kernel,
        grid=(nb,),
        in_specs=[
            pl.BlockSpec((_BM, n), lambda i: (i, 0)),
            pl.BlockSpec((n, d), lambda i: (0, 0)),
        ],
        out_specs=[
            pl.BlockSpec((1, _BM, n), lambda i: (i, 0, 0)),
            pl.BlockSpec((_BM, 1), lambda i: (i, 0)),
            pl.BlockSpec((_BM, d), lambda i: (i, 0)),
        ],
        out_shape=[
            jax.ShapeDtypeStruct((nb, _BM, n), _QA),
            jax.ShapeDtypeStruct((n, 1), _F32),
            jax.ShapeDtypeStruct((n, d), _Q),
        ],
    )(adj, hq)


_SLAB = 5


def _prop_q_kernel(aq_ref, rs_ref, hq_ref, o_ref):
    hq = hq_ref[...]
    for k in range(_SLAB):
        acc = jax.lax.dot_general(aq_ref[k], hq, (((1,), (0,)), ((), ())),
                                  preferred_element_type=_F32)
        rs = rs_ref[k * _BM:(k + 1) * _BM, :]
        o_ref[k * _BM:(k + 1) * _BM, :] = (acc * (rs * _INV)).astype(_Q)


def _prop_y_kernel(aq_ref, rs_ref, hq_ref, cs_ref, o_ref):
    hq = hq_ref[...]
    cs = cs_ref[...]
    for k in range(_SLAB):
        acc = jax.lax.dot_general(aq_ref[k], hq, (((1,), (0,)), ((), ())),
                                  preferred_element_type=_F32)
        rs = rs_ref[k * _BM:(k + 1) * _BM, :]
        o_ref[k * _BM:(k + 1) * _BM, :] = (acc * rs * cs).astype(o_ref.dtype)


def _prop(aq, rs, hq, cs=None, out_dtype=None):
    nb, bm, n = aq.shape
    d = hq.shape[1]
    in_specs = [
        pl.BlockSpec((_SLAB, bm, n), lambda i: (i, 0, 0)),
        pl.BlockSpec((_SLAB * bm, 1), lambda i: (i, 0)),
        pl.BlockSpec((n, d), lambda i: (0, 0)),
    ]
    args = [aq, rs, hq]
    if cs is None:
        body = _prop_q_kernel
        out_dtype = _Q
    else:
        body = _prop_y_kernel
        in_specs.append(pl.BlockSpec((1, d), lambda i: (0, 0)))
        args.append(cs)
    return pl.pallas_call(
        body,
        grid=(nb // _SLAB,),
        in_specs=in_specs,
        out_specs=pl.BlockSpec((_SLAB * bm, d), lambda i: (i, 0)),
        out_shape=jax.ShapeDtypeStruct((n, d), out_dtype),
    )(*args)


def _sweeps_kernel(aq_ref, rs_ref, hq0_ref, cs_ref, o_ref, h0_scr, h1_scr,
                   *, nsweeps):
    s = pl.program_id(0)
    i = pl.program_id(1)

    @pl.when(jnp.logical_and(s == 0, i == 0))
    def _():
        h1_scr[...] = hq0_ref[...]

    def body(src_scr, dst_scr):
        hq = src_scr[...]
        cs = cs_ref[...]
        for k in range(_SLAB):
            acc = jax.lax.dot_general(aq_ref[k], hq, (((1,), (0,)), ((), ())),
                                      preferred_element_type=_F32)
            rs = rs_ref[k * _BM:(k + 1) * _BM, :]

            @pl.when(s < nsweeps - 1)
            def _(acc=acc, rs=rs, k=k):
                dst_scr[pl.ds((i * _SLAB + k) * _BM, _BM), :] = (
                    acc * (rs * _INV)).astype(_Q)

            @pl.when(s == nsweeps - 1)
            def _(acc=acc, rs=rs, k=k):
                o_ref[k * _BM:(k + 1) * _BM, :] = (
                    acc * rs * cs).astype(o_ref.dtype)

    parity = jax.lax.rem(s, 2)

    @pl.when(parity == 0)
    def _():
        body(h1_scr, h0_scr)

    @pl.when(parity == 1)
    def _():
        body(h0_scr, h1_scr)


def _sweeps(aq, rs, hq, cs, nsweeps, out_dtype):
    nb, bm, n = aq.shape
    d = hq.shape[1]
    from jax.experimental.pallas import tpu as pltpu
    return pl.pallas_call(
        functools.partial(_sweeps_kernel, nsweeps=nsweeps),
        grid=(nsweeps, nb // _SLAB),
        in_specs=[
            pl.BlockSpec((_SLAB, bm, n), lambda s, i: (i, 0, 0)),
            pl.BlockSpec((_SLAB * bm, 1), lambda s, i: (i, 0)),
            pl.BlockSpec((n, d), lambda s, i: (0, 0)),
            pl.BlockSpec((1, d), lambda s, i: (0, 0)),
        ],
        out_specs=pl.BlockSpec((_SLAB * bm, d), lambda s, i: (i, 0)),
        out_shape=jax.ShapeDtypeStruct((n, d), out_dtype),
        scratch_shapes=[pltpu.VMEM((n, d), _Q), pltpu.VMEM((n, d), _Q)],
    )(aq, rs, hq, cs)


def _logsoftmax_kernel(y_ref, o_ref):
    y = y_ref[...]
    m = jnp.max(y, axis=1, keepdims=True)
    e = y - m
    lse = jnp.log(jnp.sum(jnp.exp(e), axis=1, keepdims=True))
    o_ref[...] = e - lse


def _logsoftmax(y):
    return pl.pallas_call(
        _logsoftmax_kernel,
        out_shape=jax.ShapeDtypeStruct(y.shape, _F32),
    )(y)


def kernel(node_ft, adj_mat, W1, b1, W2, b2):
    hq, cs = _xq(node_ft, W1, b1, act=None)
    aq, rs, hq = _qprop(adj_mat, hq)
    y = _sweeps(aq, rs, hq, cs * (_SAFETY ** 4), nsweeps=4, out_dtype=_BF16)
    hq, cs = _xq(y, W2, b2, act="elu")
    y = _sweeps(aq, rs, hq, cs * (_SAFETY ** 4), nsweeps=5, out_dtype=_F32)
    return _logsoftmax(y)


# speedup vs baseline: 1.3824x; 1.3824x over previous
import functools

import jax
import jax.numpy as jnp
from jax.experimental import pallas as pl

_F32 = jnp.float32
_BF16 = jnp.bfloat16
_Q = jnp.float8_e4m3fn
_QA = jnp.float4_e2m1fn

_BM = 400
_SAFETY = 1.1
_INV = 1.0 / _SAFETY


def _xq_kernel(x_ref, w_ref, b_ref, hq_ref, cs_ref, *, act):
    x = x_ref[...].astype(_F32)
    if act == "elu":
        x = jnp.where(x > 0, x, jnp.exp(x) - 1.0)
    y = jnp.dot(x.astype(_BF16), w_ref[...].astype(_BF16),
                preferred_element_type=_F32) + b_ref[...]
    cmax = jnp.maximum(jnp.max(jnp.abs(y), axis=0, keepdims=True), 1e-30)
    cs_ref[...] = cmax
    hq_ref[...] = (y * (1.0 / cmax)).astype(_Q)


def _xq(x, w, b, act):
    n, d_in = x.shape
    d_out = w.shape[1]
    return pl.pallas_call(
        functools.partial(_xq_kernel, act=act),
        out_shape=[
            jax.ShapeDtypeStruct((n, d_out), _Q),
            jax.ShapeDtypeStruct((1, d_out), _F32),
        ],
    )(x, w, b.reshape(1, d_out))


def _qprop_kernel(a_ref, hq_ref, aq_ref, rs_ref, hqn_ref):
    a = a_ref[...]
    rowmax = jnp.maximum(jnp.max(a, axis=1, keepdims=True), 1e-30)
    rs_ref[...] = rowmax
    aq = (a * (1.0 / rowmax)).astype(_QA)
    aq_ref[0] = aq
    acc = jax.lax.dot_general(aq, hq_ref[...], (((1,), (0,)), ((), ())),
                              preferred_element_type=_F32)
    hqn_ref[...] = (acc * (rowmax * _INV)).astype(_Q)


def _qprop(adj, hq):
    n = adj.shape[0]
    d = hq.shape[1]
    nb = n // _BM
    return pl.pallas_call(
        _qprop_kernel,
        grid=(nb,),
        in_specs=[
            pl.BlockSpec((_BM, n), lambda i: (i, 0)),
            pl.BlockSpec((n, d), lambda i: (0, 0)),
        ],
        out_specs=[
            pl.BlockSpec((1, _BM, n), lambda i: (i, 0, 0)),
            pl.BlockSpec((_BM, 1), lambda i: (i, 0)),
            pl.BlockSpec((_BM, d), lambda i: (i, 0)),
        ],
        out_shape=[
            jax.ShapeDtypeStruct((nb, _BM, n), _QA),
            jax.ShapeDtypeStruct((n, 1), _F32),
            jax.ShapeDtypeStruct((n, d), _Q),
        ],
    )(adj, hq)


_SLAB = 5


def _prop_q_kernel(aq_ref, rs_ref, hq_ref, o_ref):
    hq = hq_ref[...]
    for k in range(_SLAB):
        acc = jax.lax.dot_general(aq_ref[k], hq, (((1,), (0,)), ((), ())),
                                  preferred_element_type=_F32)
        rs = rs_ref[k * _BM:(k + 1) * _BM, :]
        o_ref[k * _BM:(k + 1) * _BM, :] = (acc * (rs * _INV)).astype(_Q)


def _prop_y_kernel(aq_ref, rs_ref, hq_ref, cs_ref, o_ref):
    hq = hq_ref[...]
    cs = cs_ref[...]
    for k in range(_SLAB):
        acc = jax.lax.dot_general(aq_ref[k], hq, (((1,), (0,)), ((), ())),
                                  preferred_element_type=_F32)
        rs = rs_ref[k * _BM:(k + 1) * _BM, :]
        o_ref[k * _BM:(k + 1) * _BM, :] = (acc * rs * cs).astype(o_ref.dtype)


def _prop(aq, rs, hq, cs=None, out_dtype=None):
    nb, bm, n = aq.shape
    d = hq.shape[1]
    in_specs = [
        pl.BlockSpec((_SLAB, bm, n), lambda i: (i, 0, 0)),
        pl.BlockSpec((_SLAB * bm, 1), lambda i: (i, 0)),
        pl.BlockSpec((n, d), lambda i: (0, 0)),
    ]
    args = [aq, rs, hq]
    if cs is None:
        body = _prop_q_kernel
        out_dtype = _Q
    else:
        body = _prop_y_kernel
        in_specs.append(pl.BlockSpec((1, d), lambda i: (0, 0)))
        args.append(cs)
    return pl.pallas_call(
        body,
        grid=(nb // _SLAB,),
        in_specs=in_specs,
        out_specs=pl.BlockSpec((_SLAB * bm, d), lambda i: (i, 0)),
        out_shape=jax.ShapeDtypeStruct((n, d), out_dtype),
    )(*args)


def _logsoftmax_kernel(y_ref, o_ref):
    y = y_ref[...]
    m = jnp.max(y, axis=1, keepdims=True)
    e = y - m
    lse = jnp.log(jnp.sum(jnp.exp(e), axis=1, keepdims=True))
    o_ref[...] = e - lse


def _logsoftmax(y):
    return pl.pallas_call(
        _logsoftmax_kernel,
        out_shape=jax.ShapeDtypeStruct(y.shape, _F32),
    )(y)


def kernel(node_ft, adj_mat, W1, b1, W2, b2):
    hq, cs = _xq(node_ft, W1, b1, act=None)
    aq, rs, hq = _qprop(adj_mat, hq)
    for _ in range(3):
        hq = _prop(aq, rs, hq)
    y = _prop(aq, rs, hq, cs=cs * (_SAFETY ** 4), out_dtype=_BF16)
    hq, cs = _xq(y, W2, b2, act="elu")
    for _ in range(4):
        hq = _prop(aq, rs, hq)
    y = _prop(aq, rs, hq, cs=cs * (_SAFETY ** 4), out_dtype=_F32)
    return _logsoftmax(y)
